# baseline (speedup 1.0000x reference)
import functools

import jax
import jax.numpy as jnp
from jax import lax
from jax.experimental import pallas as pl
from jax.experimental.pallas import tpu as pltpu
from jax.experimental.pallas import tpu_sc as plsc

NU = 100000
NIJK = 50000
NIJKL = 400000
NE = 1600000
PAIR = 128
H = 8
MAXPI = 8

NSUB = 16

NIJKL_PAD = 458752
NE_PAD = 1638400
NIJK_PAD = 50048
SEG_SLICE = 3128
NU_PAD = 100352
NU_TC = 100352

IJ_PER_TILE = 14336
IJ_CHUNK = 1024
IJ_NCHUNK = 14

E_PER_TILE = 51200
E_CHUNK = 512
E_NCHUNK = 100
K3ROWS = NE_PAD * 8 // 1024
KC = E_CHUNK * 8 // 1024

PIECE = 128

U_PER_TILE = 6272
U_PIECE = 896
U_NPIECE = 7

_MESH = plsc.VectorSubcoreMesh(core_axis_name="c", subcore_axis_name="s")
_SC_PARAMS = pltpu.CompilerParams(
    needs_layout_passes=False, use_tc_tiling_on_sc=False)

_f32 = jnp.float32
_i32 = jnp.int32


def _vloop(n, body):
  lax.fori_loop(0, n, lambda i, c: (body(i), 0)[1], 0)


def _iota16():
  return lax.iota(_i32, 16)


def _rc(i, iota):
  fi = i * 16 + iota
  return [jnp.right_shift(fi, 3), jnp.bitwise_and(fi, 7)]


def _hu(i, iota):
  fi = i * 16 + iota
  return [jnp.bitwise_and(fi, 7), jnp.right_shift(fi, 3)]


def _add_into(dst2d, src2d, nwords):
  iota = _iota16()

  def b(i):
    rc = _rc(i, iota)
    v = plsc.load_gather(dst2d, rc) + plsc.load_gather(src2d, rc)
    plsc.store_scatter(dst2d, rc, v)

  _vloop(nwords // 16, b)


def _fill_zero(buf2d, nwords):
  z = jnp.zeros((16,), _f32)
  iota = _iota16()

  def b(i):
    plsc.store_scatter(buf2d, _rc(i, iota), z)

  _vloop(nwords // 16, b)


def _copy_idx(idxv, idx2, nwords):
  def b(m):
    j = m // 8
    k = m - 8 * j
    idx2[j, pl.ds(k * 16, 16)] = idxv[pl.ds(m * 16, 16)]

  _vloop(nwords // 16, b)


def _t_to_rows(src8w, dst8, nwords):
  iota = _iota16()

  def b(i):
    v = plsc.load_gather(src8w, _hu(i, iota))
    plsc.store_scatter(dst8, _rc(i, iota), v)

  _vloop(nwords // 16, b)


def _tca_body(sref, wref, oref):
  i = pl.program_id(0)
  tkt = lax.dot_general(
      wref[...], sref[...], (((0,), (1,)), ((), ())),
      preferred_element_type=_f32)
  col = i * 1024 + lax.broadcasted_iota(_i32, (1, 1024), 1)
  oref[...] = jnp.where(col < NIJKL, jnp.exp(tkt), 0.0)


def _tcb_body(pref, wref, oref):
  oref[...] = lax.dot_general(
      wref[...], pref[...], (((0,), (1,)), ((), ())),
      preferred_element_type=_f32)


def _tcc_body(xref, pref, w1ref, b1ref, w2ref, b2ref, oref):
  xt = xref[...]
  h = lax.dot_general(
      xt, w1ref[...], (((0,), (0,)), ((), ())),
      preferred_element_type=_f32) + b1ref[...]
  h = jax.nn.gelu(h)
  y = jnp.dot(h, w2ref[...], preferred_element_type=_f32) + b2ref[...]
  oref[...] = y + pref[...]


def _s1_body(extk, segi, denomA, denomB, dsp, idxv, idx2, ex8, dat8, zb):
  c = lax.axis_index("c")
  s = lax.axis_index("s")
  wid = c * NSUB + s
  _fill_zero(zb, SEG_SLICE * 8)
  pltpu.sync_copy(zb, dsp.at[pl.ds(s * SEG_SLICE, SEG_SLICE), :])
  plsc.subcore_barrier()

  def chunk(i):
    r0 = wid * IJ_PER_TILE + i * IJ_CHUNK
    pltpu.sync_copy(segi.at[pl.ds(r0, IJ_CHUNK)], idxv)
    _copy_idx(idxv, idx2, IJ_CHUNK)
    pltpu.sync_copy(extk.at[:, pl.ds(r0, IJ_CHUNK)], ex8)
    _t_to_rows(ex8, dat8, IJ_CHUNK * 8)
    for j in range(IJ_CHUNK // PIECE):
      pltpu.sync_copy(
          dat8.at[pl.ds(j * PIECE, PIECE), :],
          dsp.at[idx2.at[j]],
          add=True,
      )

  _vloop(IJ_NCHUNK, chunk)
  plsc.subcore_barrier()
  pltpu.sync_copy(dsp.at[pl.ds(s * SEG_SLICE, SEG_SLICE), :], zb)

  @pl.when(c == 0)
  def _():
    pltpu.sync_copy(zb, denomA.at[pl.ds(s * SEG_SLICE, SEG_SLICE), :])

  @pl.when(c == 1)
  def _():
    pltpu.sync_copy(zb, denomB.at[pl.ds(s * SEG_SLICE, SEG_SLICE), :])


def _s2_body(extk, segi, denomA, denomB, alpha, dsp, pa, pb, idxv, idx2,
             ex8, dat8, gat):
  c = lax.axis_index("c")
  s = lax.axis_index("s")
  wid = c * NSUB + s
  sl_seg = pl.ds(s * SEG_SLICE, SEG_SLICE)
  pltpu.sync_copy(denomA.at[sl_seg, :], pa)
  pltpu.sync_copy(denomB.at[sl_seg, :], pb)
  _add_into(pa, pb, SEG_SLICE * 8)
  pltpu.sync_copy(pa, dsp.at[sl_seg, :])
  plsc.subcore_barrier()
  iota = _iota16()
  lane_row = jnp.right_shift(iota, 3)

  def chunk(i):
    r0 = wid * IJ_PER_TILE + i * IJ_CHUNK
    pltpu.sync_copy(segi.at[pl.ds(r0, IJ_CHUNK)], idxv)
    _copy_idx(idxv, idx2, IJ_CHUNK)
    pltpu.sync_copy(extk.at[:, pl.ds(r0, IJ_CHUNK)], ex8)
    _t_to_rows(ex8, dat8, IJ_CHUNK * 8)
    for j in range(IJ_CHUNK // PIECE):
      pltpu.sync_copy(
          dsp.at[idx2.at[j]], gat.at[pl.ds(j * PIECE, PIECE), :])

    def b(i2):
      rc = _rc(i2, iota)
      d = plsc.load_gather(dat8, rc)
      g = plsc.load_gather(gat, rc)
      rid = (r0 + 2 * i2) + lane_row
      m = rid < NIJKL
      plsc.store_scatter(dat8, rc, jnp.where(m, d / (g + 1e-16), 0.0))

    _vloop(IJ_CHUNK // 2, b)
    pltpu.sync_copy(dat8, alpha.at[pl.ds(r0, IJ_CHUNK), :])

  _vloop(IJ_NCHUNK, chunk)


def _s3_body(gi, alpha, kern, idx2, gat, gat2):
  c = lax.axis_index("c")
  s = lax.axis_index("s")
  wid = c * NSUB + s
  iota = _iota16()
  npc = E_CHUNK // PIECE

  def chunk(i):
    r4 = wid * (E_PER_TILE // PIECE) + i * npc
    pltpu.sync_copy(gi.at[pl.ds(r4, npc), :], idx2)
    for j in range(npc):
      pltpu.sync_copy(
          alpha.at[idx2.at[j]], gat.at[pl.ds(j * PIECE, PIECE), :])

    def b(m):
      t = m // 64
      r = (m // 8) % 8
      k = m - 8 * (m // 8)
      gat2[t, r, pl.ds(k * 16, 16)] = plsc.load_gather(gat, _rc(m, iota))

    _vloop(E_CHUNK * 8 // 16, b)
    pltpu.sync_copy(gat2, kern.at[pl.ds(r4, KC), :, :])

  _vloop(E_NCHUNK, chunk)


def _step_body(t, X, *rest):
  if t > 1:
    (Bin, idxU, idxu, kern, Bout, tsp, asp, xt, ad, idx2U0, idx2U1, idx2u0,
     idx2u1, kv0, kv1, gb, insem0, insem1, gsem, gsem2, ssem) = rest
  else:
    (idxU, idxu, kern, Bout, tsp, asp, xt, ad, idx2U0, idx2U1, idx2u0,
     idx2u1, kv0, kv1, gb, insem0, insem1, gsem, gsem2, ssem) = rest
    Bin = None
  idx2U = [idx2U0, idx2U1]
  idx2u = [idx2u0, idx2u1]
  kv = [kv0, kv1]
  insem = [insem0, insem1]
  c = lax.axis_index("c")
  s = lax.axis_index("s")
  wid = c * NSUB + s
  rowp = 8 * (t - 1)
  rowt = 8 * t
  iota = _iota16()
  z16 = jnp.zeros((16,), _f32)

  for p in range(U_NPIECE):
    u0 = s * U_PER_TILE + p * U_PIECE
    pltpu.sync_copy(X.at[pl.ds(rowp, 8), pl.ds(u0, U_PIECE)], xt)
    if t > 1:
      pltpu.sync_copy(Bin.at[pl.ds(u0, U_PIECE), :], ad)

    def b(i):
      v = plsc.load_gather(xt, _hu(i, iota))
      if t > 1:
        v = v + plsc.load_gather(ad, _rc(i, iota))
        plsc.store_scatter(xt, _hu(i, iota), v)
      plsc.store_scatter(ad, _rc(i, iota), v)

    _vloop(U_PIECE * 8 // 16, b)
    pltpu.sync_copy(ad, tsp.at[pl.ds(u0, U_PIECE), :])
    if t > 1:
      @pl.when(c == 0)
      def _():
        pltpu.sync_copy(xt, X.at[pl.ds(rowp, 8), pl.ds(u0, U_PIECE)])
    _fill_zero(ad, U_PIECE * 8)
    pltpu.sync_copy(ad, asp.at[pl.ds(u0, U_PIECE), :])
  plsc.subcore_barrier()

  npc = E_CHUNK // PIECE

  def _in_descs(i, b):
    r8 = wid * (E_PER_TILE // PIECE) + i * npc
    return [
        pltpu.make_async_copy(
            idxU.at[pl.ds(r8, npc), :], idx2U[b], insem[b]),
        pltpu.make_async_copy(
            idxu.at[pl.ds(r8, npc), :], idx2u[b], insem[b]),
        pltpu.make_async_copy(
            kern.at[pl.ds(r8 * (KC // npc), KC), :, :], kv[b], insem[b]),
    ]

  def _sc_descs(b):
    return [
        pltpu.make_async_copy(
            gb.at[pl.ds(j * PIECE, PIECE), :], asp.at[idx2u[b].at[j]], ssem)
        for j in range(npc)
    ]

  for d in _in_descs(0, 0):
    d.start()

  def chunk(it):
    for b in range(2):
      i = it * 2 + b

      @pl.when(i + 1 < E_NCHUNK)
      def _():
        for d in _in_descs(i + 1, 1 - b):
          d.start()

      for d in _in_descs(i, b):
        d.wait()
      gds = [
          pltpu.make_async_copy(
              tsp.at[idx2U[b].at[j]], gb.at[pl.ds(j * PIECE, PIECE), :], gsem)
          for j in range(npc)
      ]
      gds[0].start()
      for j in range(npc):
        gds[j].wait()
        if j + 1 < npc:
          gds[j + 1].start()

      def bmul(mm):
        t = mm // 64
        r = (mm // 8) % 8
        k = mm - 8 * (mm // 8)
        rc = _rc(mm, iota)
        v = plsc.load_gather(gb, rc) * kv[b][t, r, pl.ds(k * 16, 16)]
        plsc.store_scatter(gb, rc, v)

      _vloop(E_CHUNK * 8 // 16, bmul)
      for d in _sc_descs(b):
        d.start(add=True)
      for d in _sc_descs(b):
        d.wait()

  _vloop(E_NCHUNK // 2, chunk)
  plsc.subcore_barrier()

  for p in range(U_NPIECE):
    u0 = s * U_PER_TILE + p * U_PIECE
    pltpu.sync_copy(asp.at[pl.ds(u0, U_PIECE), :], ad)

    @pl.when(c == 0)
    def _():
      for h in range(8):
        hv = jnp.full((16,), h, _i32)

        def b2(k):
          v = plsc.load_gather(ad, [k * 16 + iota, hv])
          xt[h, pl.ds(k * 16, 16)] = v

        _vloop(U_PIECE // 16, b2)
      pltpu.sync_copy(xt, X.at[pl.ds(rowt, 8), pl.ds(u0, U_PIECE)])

    @pl.when(c == 1)
    def _():
      pltpu.sync_copy(ad, Bout.at[pl.ds(u0, U_PIECE), :])


def _s4_body(X, B9, xt, ad):
  c = lax.axis_index("c")
  s = lax.axis_index("s")
  iota = _iota16()

  @pl.when(c == 0)
  def _():
    for p in range(U_NPIECE):
      u0 = s * U_PER_TILE + p * U_PIECE
      pltpu.sync_copy(X.at[pl.ds(64, 8), pl.ds(u0, U_PIECE)], xt)
      pltpu.sync_copy(B9.at[pl.ds(u0, U_PIECE), :], ad)

      def b(i):
        v = (plsc.load_gather(xt, _hu(i, iota))
             + plsc.load_gather(ad, _rc(i, iota)))
        plsc.store_scatter(xt, _hu(i, iota), v)

      _vloop(U_PIECE * 8 // 16, b)
      pltpu.sync_copy(xt, X.at[pl.ds(64, 8), pl.ds(u0, U_PIECE)])


_s1 = pl.kernel(
    _s1_body,
    out_type=(jax.ShapeDtypeStruct((NIJK_PAD, 8), _f32),) * 2,
    mesh=_MESH,
    compiler_params=_SC_PARAMS,
    scratch_types=[
        pltpu.MemorySpace.VMEM_SHARED((NIJK_PAD, 8), _f32),
        pltpu.VMEM((IJ_CHUNK,), _i32),
        pltpu.VMEM((IJ_CHUNK // PIECE, PIECE), _i32),
        pltpu.VMEM((8, IJ_CHUNK), _f32),
        pltpu.VMEM((IJ_CHUNK, 8), _f32),
        pltpu.VMEM((SEG_SLICE, 8), _f32),
    ],
)

_s2 = pl.kernel(
    _s2_body,
    out_type=jax.ShapeDtypeStruct((NIJKL_PAD, 8), _f32),
    mesh=_MESH,
    compiler_params=_SC_PARAMS,
    scratch_types=[
        pltpu.MemorySpace.VMEM_SHARED((NIJK_PAD, 8), _f32),
        pltpu.VMEM((SEG_SLICE, 8), _f32),
        pltpu.VMEM((SEG_SLICE, 8), _f32),
        pltpu.VMEM((IJ_CHUNK,), _i32),
        pltpu.VMEM((IJ_CHUNK // PIECE, PIECE), _i32),
        pltpu.VMEM((8, IJ_CHUNK), _f32),
        pltpu.VMEM((IJ_CHUNK, 8), _f32),
        pltpu.VMEM((IJ_CHUNK, 8), _f32),
    ],
)

_s3 = pl.kernel(
    _s3_body,
    out_type=jax.ShapeDtypeStruct((K3ROWS, 8, 128), _f32),
    mesh=_MESH,
    compiler_params=_SC_PARAMS,
    scratch_types=[
        pltpu.VMEM((E_CHUNK // PIECE, PIECE), _i32),
        pltpu.VMEM((E_CHUNK, 8), _f32),
        pltpu.VMEM((KC, 8, 128), _f32),
    ],
)

_step_scratch = [
    pltpu.MemorySpace.VMEM_SHARED((NU_PAD, 8), _f32),
    pltpu.MemorySpace.VMEM_SHARED((NU_PAD, 8), _f32),
    pltpu.VMEM((8, U_PIECE), _f32),
    pltpu.VMEM((U_PIECE, 8), _f32),
    pltpu.VMEM((E_CHUNK // PIECE, PIECE), _i32),
    pltpu.VMEM((E_CHUNK // PIECE, PIECE), _i32),
    pltpu.VMEM((E_CHUNK // PIECE, PIECE), _i32),
    pltpu.VMEM((E_CHUNK // PIECE, PIECE), _i32),
    pltpu.VMEM((KC, 8, 128), _f32),
    pltpu.VMEM((KC, 8, 128), _f32),
    pltpu.VMEM((E_CHUNK, 8), _f32),
    pltpu.SemaphoreType.DMA,
    pltpu.SemaphoreType.DMA,
    pltpu.SemaphoreType.DMA,
    pltpu.SemaphoreType.DMA,
    pltpu.SemaphoreType.DMA,
]

_steps = [
    pl.kernel(
        functools.partial(_step_body, t),
        out_type=jax.ShapeDtypeStruct((NU_PAD, 8), _f32),
        mesh=_MESH,
        compiler_params=_SC_PARAMS,
        scratch_types=list(_step_scratch),
    )
    for t in range(1, MAXPI + 1)
]

_s4 = pl.kernel(
    _s4_body,
    out_type=(),
    mesh=_MESH,
    compiler_params=_SC_PARAMS,
    scratch_types=[
        pltpu.VMEM((8, U_PIECE), _f32),
        pltpu.VMEM((U_PIECE, 8), _f32),
    ],
)

_tca = pl.pallas_call(
    _tca_body,
    grid=(NIJKL_PAD // 1024,),
    in_specs=[
        pl.BlockSpec((1024, PAIR), lambda i: (jnp.minimum(i, NIJKL // 1024), 0)),
        pl.BlockSpec((PAIR, H), lambda i: (0, 0)),
    ],
    out_specs=pl.BlockSpec((H, 1024), lambda i: (0, i)),
    out_shape=jax.ShapeDtypeStruct((H, NIJKL_PAD), _f32),
)

_tcb = pl.pallas_call(
    _tcb_body,
    grid=(NU_TC // 1024,),
    in_specs=[
        pl.BlockSpec((1024, PAIR), lambda i: (jnp.minimum(i, NU // 1024), 0)),
        pl.BlockSpec((PAIR, PAIR), lambda i: (0, 0)),
    ],
    out_specs=pl.BlockSpec((PAIR, 1024), lambda i: (0, i)),
    out_shape=jax.ShapeDtypeStruct((PAIR, NU_PAD), _f32),
)

_tcc = pl.pallas_call(
    _tcc_body,
    grid=(NU_TC // 1024,),
    in_specs=[
        pl.BlockSpec((PAIR, 1024), lambda i: (0, i)),
        pl.BlockSpec((1024, PAIR), lambda i: (jnp.minimum(i, NU // 1024), 0)),
        pl.BlockSpec((PAIR, PAIR), lambda i: (0, 0)),
        pl.BlockSpec((1, PAIR), lambda i: (0, 0)),
        pl.BlockSpec((PAIR, PAIR), lambda i: (0, 0)),
        pl.BlockSpec((1, PAIR), lambda i: (0, 0)),
    ],
    out_specs=pl.BlockSpec((1024, PAIR), lambda i: (i, 0)),
    out_shape=jax.ShapeDtypeStruct((NU, PAIR), _f32),
)


def kernel(prop_attr, stereo_attr, gather_idx_ijkl_jkl, gather_idx_Uijkl_ijkl,
           gather_idx_Uijkl_Uijk, gather_idx_Uijkl_ujkl, num_ijk, num_Uijk,
           W_value, W_kernel, W1, b1, W2, b2):
  segi = jnp.concatenate([
      gather_idx_ijkl_jkl.astype(_i32),
      jnp.zeros((NIJKL_PAD - NIJKL,), _i32),
  ])
  gi = jnp.concatenate([
      gather_idx_Uijkl_ijkl.astype(_i32),
      jnp.full((NE_PAD - NE,), NIJKL, _i32),
  ]).reshape(NE_PAD // PIECE, PIECE)
  idxU = jnp.concatenate([
      gather_idx_Uijkl_Uijk.astype(_i32),
      jnp.zeros((NE_PAD - NE,), _i32),
  ]).reshape(NE_PAD // PIECE, PIECE)
  idxu = jnp.concatenate([
      gather_idx_Uijkl_ujkl.astype(_i32),
      jnp.zeros((NE_PAD - NE,), _i32),
  ]).reshape(NE_PAD // PIECE, PIECE)
  Wv_pad = jnp.concatenate(
      [W_value, jnp.zeros((PAIR, PAIR - H), _f32)], axis=1)
  W1p = jnp.concatenate(
      [W1, jnp.zeros((PAIR - H * (MAXPI + 1), PAIR), _f32)], axis=0)
  b1r = b1.reshape(1, PAIR)
  b2r = b2.reshape(1, PAIR)

  extk = _tca(stereo_attr, W_kernel)
  denomA, denomB = _s1(extk, segi)
  alpha = _s2(extk, segi, denomA, denomB)
  kern = _s3(gi, alpha)

  X = _tcb(prop_attr, Wv_pad)
  Xr = jax.new_ref(X)
  B = _steps[0](Xr, idxU, idxu, kern)
  for t in range(2, MAXPI + 1):
    B = _steps[t - 1](Xr, B, idxU, idxu, kern)
  _s4(Xr, B)
  Xf = Xr[...]

  return _tcc(Xf, prop_attr, W1p, b1r, W2, b2r)


# speedup vs baseline: 1.0959x; 1.0959x over previous
import functools

import jax
import jax.numpy as jnp
from jax import lax
from jax.experimental import pallas as pl
from jax.experimental.pallas import tpu as pltpu
from jax.experimental.pallas import tpu_sc as plsc

NU = 100000
NIJK = 50000
NIJKL = 400000
NE = 1600000
PAIR = 128
H = 8
MAXPI = 8

NSUB = 16

NIJKL_PAD = 458752
NE_PAD = 1638400
NIJK_PAD = 50048
SEG_SLICE = 3128
NU_PAD = 100352
NU_TC = 100352

IJ_PER_TILE = 14336
IJ_CHUNK = 1024
IJ_NCHUNK = 14

E_PER_TILE = 51200
E_CHUNK = 512
E_NCHUNK = 100
K3ROWS = NE_PAD * 8 // 1024
KC = E_CHUNK * 8 // 1024

PIECE = 128

U_PER_TILE = 6272
U_PIECE = 896
U_NPIECE = 7

_MESH = plsc.VectorSubcoreMesh(core_axis_name="c", subcore_axis_name="s")
_SC_PARAMS = pltpu.CompilerParams(
    needs_layout_passes=False, use_tc_tiling_on_sc=False)

_f32 = jnp.float32
_i32 = jnp.int32


def _vloop(n, body):
  lax.fori_loop(0, n, lambda i, c: (body(i), 0)[1], 0)


def _iota16():
  return lax.iota(_i32, 16)


def _rc(i, iota):
  fi = i * 16 + iota
  return [jnp.right_shift(fi, 3), jnp.bitwise_and(fi, 7)]


def _hu(i, iota):
  fi = i * 16 + iota
  return [jnp.bitwise_and(fi, 7), jnp.right_shift(fi, 3)]


def _add_into(dst2d, src2d, nwords):
  iota = _iota16()

  def b(i):
    rc = _rc(i, iota)
    v = plsc.load_gather(dst2d, rc) + plsc.load_gather(src2d, rc)
    plsc.store_scatter(dst2d, rc, v)

  _vloop(nwords // 16, b)


def _fill_zero(buf2d, nwords):
  z = jnp.zeros((16,), _f32)
  iota = _iota16()

  def b(i):
    plsc.store_scatter(buf2d, _rc(i, iota), z)

  _vloop(nwords // 16, b)


def _copy_idx(idxv, idx2, nwords):
  def b(m):
    j = m // 8
    k = m - 8 * j
    idx2[j, pl.ds(k * 16, 16)] = idxv[pl.ds(m * 16, 16)]

  _vloop(nwords // 16, b)


def _t_to_rows(src8w, dst8, nwords):
  iota = _iota16()

  def b(i):
    v = plsc.load_gather(src8w, _hu(i, iota))
    plsc.store_scatter(dst8, _rc(i, iota), v)

  _vloop(nwords // 16, b)


def _tca_body(sref, wref, oref):
  i = pl.program_id(0)
  tkt = lax.dot_general(
      wref[...], sref[...], (((0,), (1,)), ((), ())),
      preferred_element_type=_f32)
  col = i * 1024 + lax.broadcasted_iota(_i32, (1, 1024), 1)
  oref[...] = jnp.where(col < NIJKL, jnp.exp(tkt), 0.0)


def _tcb_body(pref, wref, oref):
  oref[...] = lax.dot_general(
      wref[...], pref[...], (((0,), (1,)), ((), ())),
      preferred_element_type=_f32)


def _tcc_body(xref, pref, w1ref, b1ref, w2ref, b2ref, oref):
  xt = xref[...]
  h = lax.dot_general(
      xt, w1ref[...], (((0,), (0,)), ((), ())),
      preferred_element_type=_f32) + b1ref[...]
  h = jax.nn.gelu(h)
  y = jnp.dot(h, w2ref[...], preferred_element_type=_f32) + b2ref[...]
  oref[...] = y + pref[...]


def _s1_body(extk, segi, denomA, denomB, dsp, idxv, idx2, ex8, dat8, zb):
  c = lax.axis_index("c")
  s = lax.axis_index("s")
  wid = c * NSUB + s
  _fill_zero(zb, SEG_SLICE * 8)
  pltpu.sync_copy(zb, dsp.at[pl.ds(s * SEG_SLICE, SEG_SLICE), :])
  plsc.subcore_barrier()

  def chunk(i):
    r0 = wid * IJ_PER_TILE + i * IJ_CHUNK
    pltpu.sync_copy(segi.at[pl.ds(r0, IJ_CHUNK)], idxv)
    _copy_idx(idxv, idx2, IJ_CHUNK)
    pltpu.sync_copy(extk.at[:, pl.ds(r0, IJ_CHUNK)], ex8)
    _t_to_rows(ex8, dat8, IJ_CHUNK * 8)
    for j in range(IJ_CHUNK // PIECE):
      pltpu.sync_copy(
          dat8.at[pl.ds(j * PIECE, PIECE), :],
          dsp.at[idx2.at[j]],
          add=True,
      )

  _vloop(IJ_NCHUNK, chunk)
  plsc.subcore_barrier()
  pltpu.sync_copy(dsp.at[pl.ds(s * SEG_SLICE, SEG_SLICE), :], zb)

  @pl.when(c == 0)
  def _():
    pltpu.sync_copy(zb, denomA.at[pl.ds(s * SEG_SLICE, SEG_SLICE), :])

  @pl.when(c == 1)
  def _():
    pltpu.sync_copy(zb, denomB.at[pl.ds(s * SEG_SLICE, SEG_SLICE), :])


def _s2_body(extk, segi, denomA, denomB, alpha, dsp, pa, pb, idxv, idx2,
             ex8, dat8, gat):
  c = lax.axis_index("c")
  s = lax.axis_index("s")
  wid = c * NSUB + s
  sl_seg = pl.ds(s * SEG_SLICE, SEG_SLICE)
  pltpu.sync_copy(denomA.at[sl_seg, :], pa)
  pltpu.sync_copy(denomB.at[sl_seg, :], pb)
  _add_into(pa, pb, SEG_SLICE * 8)
  pltpu.sync_copy(pa, dsp.at[sl_seg, :])
  plsc.subcore_barrier()
  iota = _iota16()
  lane_row = jnp.right_shift(iota, 3)

  def chunk(i):
    r0 = wid * IJ_PER_TILE + i * IJ_CHUNK
    pltpu.sync_copy(segi.at[pl.ds(r0, IJ_CHUNK)], idxv)
    _copy_idx(idxv, idx2, IJ_CHUNK)
    pltpu.sync_copy(extk.at[:, pl.ds(r0, IJ_CHUNK)], ex8)
    _t_to_rows(ex8, dat8, IJ_CHUNK * 8)
    for j in range(IJ_CHUNK // PIECE):
      pltpu.sync_copy(
          dsp.at[idx2.at[j]], gat.at[pl.ds(j * PIECE, PIECE), :])

    def b(i2):
      rc = _rc(i2, iota)
      d = plsc.load_gather(dat8, rc)
      g = plsc.load_gather(gat, rc)
      rid = (r0 + 2 * i2) + lane_row
      m = rid < NIJKL
      plsc.store_scatter(dat8, rc, jnp.where(m, d / (g + 1e-16), 0.0))

    _vloop(IJ_CHUNK // 2, b)
    pltpu.sync_copy(dat8, alpha.at[pl.ds(r0, IJ_CHUNK), :])

  _vloop(IJ_NCHUNK, chunk)


def _s3_body(gi, alpha, kern, idx2, gat, gat2):
  c = lax.axis_index("c")
  s = lax.axis_index("s")
  wid = c * NSUB + s
  iota = _iota16()
  npc = E_CHUNK // PIECE

  def chunk(i):
    r4 = wid * (E_PER_TILE // PIECE) + i * npc
    pltpu.sync_copy(gi.at[pl.ds(r4, npc), :], idx2)
    for j in range(npc):
      pltpu.sync_copy(
          alpha.at[idx2.at[j]], gat.at[pl.ds(j * PIECE, PIECE), :])

    def b(m):
      t = m // 64
      r = (m // 8) % 8
      k = m - 8 * (m // 8)
      gat2[t, r, pl.ds(k * 16, 16)] = plsc.load_gather(gat, _rc(m, iota))

    _vloop(E_CHUNK * 8 // 16, b)
    pltpu.sync_copy(gat2, kern.at[pl.ds(r4, KC), :, :])

  _vloop(E_NCHUNK, chunk)


def _step_body(t, X, *rest):
  if t > 1:
    (Bin, idxU, idxu, kern, Bout, tsp, asp, xt, ad, idx2U0, idx2U1, idx2u0,
     idx2u1, kv0, kv1, gb, insem0, insem1, gsem, gsem2, ssem) = rest
  else:
    (idxU, idxu, kern, Bout, tsp, asp, xt, ad, idx2U0, idx2U1, idx2u0,
     idx2u1, kv0, kv1, gb, insem0, insem1, gsem, gsem2, ssem) = rest
    Bin = None
  idx2U = [idx2U0, idx2U1]
  idx2u = [idx2u0, idx2u1]
  kv = [kv0, kv1]
  insem = [insem0, insem1]
  c = lax.axis_index("c")
  s = lax.axis_index("s")
  wid = c * NSUB + s
  rowp = 8 * (t - 1)
  rowt = 8 * t
  iota = _iota16()
  z16 = jnp.zeros((16,), _f32)

  for p in range(U_NPIECE):
    u0 = s * U_PER_TILE + p * U_PIECE
    pltpu.sync_copy(X.at[pl.ds(rowp, 8), pl.ds(u0, U_PIECE)], xt)
    if t > 1:
      pltpu.sync_copy(Bin.at[pl.ds(u0, U_PIECE), :], ad)

    def b(i):
      v = plsc.load_gather(xt, _hu(i, iota))
      if t > 1:
        v = v + plsc.load_gather(ad, _rc(i, iota))
        plsc.store_scatter(xt, _hu(i, iota), v)
      plsc.store_scatter(ad, _rc(i, iota), v)

    _vloop(U_PIECE * 8 // 16, b)
    pltpu.sync_copy(ad, tsp.at[pl.ds(u0, U_PIECE), :])
    if t > 1:
      @pl.when(c == 0)
      def _():
        pltpu.sync_copy(xt, X.at[pl.ds(rowp, 8), pl.ds(u0, U_PIECE)])
    _fill_zero(ad, U_PIECE * 8)
    pltpu.sync_copy(ad, asp.at[pl.ds(u0, U_PIECE), :])
  plsc.subcore_barrier()

  npc = E_CHUNK // PIECE

  def _in_descs(i, b):
    r8 = wid * (E_PER_TILE // PIECE) + i * npc
    return [
        pltpu.make_async_copy(
            idxU.at[pl.ds(r8, npc), :], idx2U[b], insem[b]),
        pltpu.make_async_copy(
            idxu.at[pl.ds(r8, npc), :], idx2u[b], insem[b]),
        pltpu.make_async_copy(
            kern.at[pl.ds(r8 * (KC // npc), KC), :, :], kv[b], insem[b]),
    ]

  def _sc_descs(b):
    return [
        pltpu.make_async_copy(
            gb.at[pl.ds(j * PIECE, PIECE), :], asp.at[idx2u[b].at[j]], ssem)
        for j in range(npc)
    ]

  for d in _in_descs(0, 0):
    d.start()

  def chunk(it):
    for b in range(2):
      i = it * 2 + b

      @pl.when(i + 1 < E_NCHUNK)
      def _():
        for d in _in_descs(i + 1, 1 - b):
          d.start()

      for d in _in_descs(i, b):
        d.wait()
      gds = [
          pltpu.make_async_copy(
              tsp.at[idx2U[b].at[j]], gb.at[pl.ds(j * PIECE, PIECE), :],
              gsem if j % 2 == 0 else gsem2)
          for j in range(npc)
      ]
      gds[0].start()
      gds[1].start()
      for j in range(npc):
        gds[j].wait()
        if j + 2 < npc:
          gds[j + 2].start()
        vpp = PIECE * 8 // 16

        def bmul(m):
          mm = j * vpp + m
          t = mm // 64
          r = (mm // 8) % 8
          k = mm - 8 * (mm // 8)
          rc = _rc(mm, iota)
          v = plsc.load_gather(gb, rc) * kv[b][t, r, pl.ds(k * 16, 16)]
          plsc.store_scatter(gb, rc, v)

        _vloop(vpp, bmul)
      for d in _sc_descs(b):
        d.start(add=True)
      for d in _sc_descs(b):
        d.wait()

  _vloop(E_NCHUNK // 2, chunk)
  plsc.subcore_barrier()

  for p in range(U_NPIECE):
    u0 = s * U_PER_TILE + p * U_PIECE
    pltpu.sync_copy(asp.at[pl.ds(u0, U_PIECE), :], ad)

    @pl.when(c == 0)
    def _():
      for h in range(8):
        hv = jnp.full((16,), h, _i32)

        def b2(k):
          v = plsc.load_gather(ad, [k * 16 + iota, hv])
          xt[h, pl.ds(k * 16, 16)] = v

        _vloop(U_PIECE // 16, b2)
      pltpu.sync_copy(xt, X.at[pl.ds(rowt, 8), pl.ds(u0, U_PIECE)])

    @pl.when(c == 1)
    def _():
      pltpu.sync_copy(ad, Bout.at[pl.ds(u0, U_PIECE), :])


def _s4_body(X, B9, xt, ad):
  c = lax.axis_index("c")
  s = lax.axis_index("s")
  iota = _iota16()

  @pl.when(c == 0)
  def _():
    for p in range(U_NPIECE):
      u0 = s * U_PER_TILE + p * U_PIECE
      pltpu.sync_copy(X.at[pl.ds(64, 8), pl.ds(u0, U_PIECE)], xt)
      pltpu.sync_copy(B9.at[pl.ds(u0, U_PIECE), :], ad)

      def b(i):
        v = (plsc.load_gather(xt, _hu(i, iota))
             + plsc.load_gather(ad, _rc(i, iota)))
        plsc.store_scatter(xt, _hu(i, iota), v)

      _vloop(U_PIECE * 8 // 16, b)
      pltpu.sync_copy(xt, X.at[pl.ds(64, 8), pl.ds(u0, U_PIECE)])


_s1 = pl.kernel(
    _s1_body,
    out_type=(jax.ShapeDtypeStruct((NIJK_PAD, 8), _f32),) * 2,
    mesh=_MESH,
    compiler_params=_SC_PARAMS,
    scratch_types=[
        pltpu.MemorySpace.VMEM_SHARED((NIJK_PAD, 8), _f32),
        pltpu.VMEM((IJ_CHUNK,), _i32),
        pltpu.VMEM((IJ_CHUNK // PIECE, PIECE), _i32),
        pltpu.VMEM((8, IJ_CHUNK), _f32),
        pltpu.VMEM((IJ_CHUNK, 8), _f32),
        pltpu.VMEM((SEG_SLICE, 8), _f32),
    ],
)

_s2 = pl.kernel(
    _s2_body,
    out_type=jax.ShapeDtypeStruct((NIJKL_PAD, 8), _f32),
    mesh=_MESH,
    compiler_params=_SC_PARAMS,
    scratch_types=[
        pltpu.MemorySpace.VMEM_SHARED((NIJK_PAD, 8), _f32),
        pltpu.VMEM((SEG_SLICE, 8), _f32),
        pltpu.VMEM((SEG_SLICE, 8), _f32),
        pltpu.VMEM((IJ_CHUNK,), _i32),
        pltpu.VMEM((IJ_CHUNK // PIECE, PIECE), _i32),
        pltpu.VMEM((8, IJ_CHUNK), _f32),
        pltpu.VMEM((IJ_CHUNK, 8), _f32),
        pltpu.VMEM((IJ_CHUNK, 8), _f32),
    ],
)

_s3 = pl.kernel(
    _s3_body,
    out_type=jax.ShapeDtypeStruct((K3ROWS, 8, 128), _f32),
    mesh=_MESH,
    compiler_params=_SC_PARAMS,
    scratch_types=[
        pltpu.VMEM((E_CHUNK // PIECE, PIECE), _i32),
        pltpu.VMEM((E_CHUNK, 8), _f32),
        pltpu.VMEM((KC, 8, 128), _f32),
    ],
)

_step_scratch = [
    pltpu.MemorySpace.VMEM_SHARED((NU_PAD, 8), _f32),
    pltpu.MemorySpace.VMEM_SHARED((NU_PAD, 8), _f32),
    pltpu.VMEM((8, U_PIECE), _f32),
    pltpu.VMEM((U_PIECE, 8), _f32),
    pltpu.VMEM((E_CHUNK // PIECE, PIECE), _i32),
    pltpu.VMEM((E_CHUNK // PIECE, PIECE), _i32),
    pltpu.VMEM((E_CHUNK // PIECE, PIECE), _i32),
    pltpu.VMEM((E_CHUNK // PIECE, PIECE), _i32),
    pltpu.VMEM((KC, 8, 128), _f32),
    pltpu.VMEM((KC, 8, 128), _f32),
    pltpu.VMEM((E_CHUNK, 8), _f32),
    pltpu.SemaphoreType.DMA,
    pltpu.SemaphoreType.DMA,
    pltpu.SemaphoreType.DMA,
    pltpu.SemaphoreType.DMA,
    pltpu.SemaphoreType.DMA,
]

_steps = [
    pl.kernel(
        functools.partial(_step_body, t),
        out_type=jax.ShapeDtypeStruct((NU_PAD, 8), _f32),
        mesh=_MESH,
        compiler_params=_SC_PARAMS,
        scratch_types=list(_step_scratch),
    )
    for t in range(1, MAXPI + 1)
]

_s4 = pl.kernel(
    _s4_body,
    out_type=(),
    mesh=_MESH,
    compiler_params=_SC_PARAMS,
    scratch_types=[
        pltpu.VMEM((8, U_PIECE), _f32),
        pltpu.VMEM((U_PIECE, 8), _f32),
    ],
)

_tca = pl.pallas_call(
    _tca_body,
    grid=(NIJKL_PAD // 1024,),
    in_specs=[
        pl.BlockSpec((1024, PAIR), lambda i: (jnp.minimum(i, NIJKL // 1024), 0)),
        pl.BlockSpec((PAIR, H), lambda i: (0, 0)),
    ],
    out_specs=pl.BlockSpec((H, 1024), lambda i: (0, i)),
    out_shape=jax.ShapeDtypeStruct((H, NIJKL_PAD), _f32),
)

_tcb = pl.pallas_call(
    _tcb_body,
    grid=(NU_TC // 1024,),
    in_specs=[
        pl.BlockSpec((1024, PAIR), lambda i: (jnp.minimum(i, NU // 1024), 0)),
        pl.BlockSpec((PAIR, PAIR), lambda i: (0, 0)),
    ],
    out_specs=pl.BlockSpec((PAIR, 1024), lambda i: (0, i)),
    out_shape=jax.ShapeDtypeStruct((PAIR, NU_PAD), _f32),
)

_tcc = pl.pallas_call(
    _tcc_body,
    grid=(NU_TC // 1024,),
    in_specs=[
        pl.BlockSpec((PAIR, 1024), lambda i: (0, i)),
        pl.BlockSpec((1024, PAIR), lambda i: (jnp.minimum(i, NU // 1024), 0)),
        pl.BlockSpec((PAIR, PAIR), lambda i: (0, 0)),
        pl.BlockSpec((1, PAIR), lambda i: (0, 0)),
        pl.BlockSpec((PAIR, PAIR), lambda i: (0, 0)),
        pl.BlockSpec((1, PAIR), lambda i: (0, 0)),
    ],
    out_specs=pl.BlockSpec((1024, PAIR), lambda i: (i, 0)),
    out_shape=jax.ShapeDtypeStruct((NU, PAIR), _f32),
)


def kernel(prop_attr, stereo_attr, gather_idx_ijkl_jkl, gather_idx_Uijkl_ijkl,
           gather_idx_Uijkl_Uijk, gather_idx_Uijkl_ujkl, num_ijk, num_Uijk,
           W_value, W_kernel, W1, b1, W2, b2):
  segi = jnp.concatenate([
      gather_idx_ijkl_jkl.astype(_i32),
      jnp.zeros((NIJKL_PAD - NIJKL,), _i32),
  ])
  gi = jnp.concatenate([
      gather_idx_Uijkl_ijkl.astype(_i32),
      jnp.full((NE_PAD - NE,), NIJKL, _i32),
  ]).reshape(NE_PAD // PIECE, PIECE)
  idxU = jnp.concatenate([
      gather_idx_Uijkl_Uijk.astype(_i32),
      jnp.zeros((NE_PAD - NE,), _i32),
  ]).reshape(NE_PAD // PIECE, PIECE)
  idxu = jnp.concatenate([
      gather_idx_Uijkl_ujkl.astype(_i32),
      jnp.zeros((NE_PAD - NE,), _i32),
  ]).reshape(NE_PAD // PIECE, PIECE)
  Wv_pad = jnp.concatenate(
      [W_value, jnp.zeros((PAIR, PAIR - H), _f32)], axis=1)
  W1p = jnp.concatenate(
      [W1, jnp.zeros((PAIR - H * (MAXPI + 1), PAIR), _f32)], axis=0)
  b1r = b1.reshape(1, PAIR)
  b2r = b2.reshape(1, PAIR)

  extk = _tca(stereo_attr, W_kernel)
  denomA, denomB = _s1(extk, segi)
  alpha = _s2(extk, segi, denomA, denomB)
  kern = _s3(gi, alpha)

  X = _tcb(prop_attr, Wv_pad)
  Xr = jax.new_ref(X)
  B = _steps[0](Xr, idxU, idxu, kern)
  for t in range(2, MAXPI + 1):
    B = _steps[t - 1](Xr, B, idxU, idxu, kern)
  _s4(Xr, B)
  Xf = Xr[...]

  return _tcc(Xf, prop_attr, W1p, b1r, W2, b2r)
